# baseline (speedup 1.0000x reference)
import functools

import jax
import jax.numpy as jnp
from jax import lax
from jax.experimental import pallas as pl
from jax.experimental.pallas import tpu as pltpu
from jax.experimental.pallas import tpu_sc as plsc

_N_TOKENS = 8192
_N_CLUSTER = 1024
_EMB = 64

_BT = 4096
_NB = _N_TOKENS // _BT
_NSTRIP = _N_CLUSTER // 128
_NCH = 8
_C = _BT // _NCH

_NC = 2
_NS = 16
_NW = _NC * _NS
_B1 = _N_TOKENS // _NW
_B2 = _N_CLUSTER // _NW
_CHUNK = 32

def _dist_argmin_kernel(e_ref, r_ref, idxe_ref, idxr_ref, cv_ref, ci_ref):
    i = pl.program_id(0)
    _BIG = jnp.float32(1e9)
    f32 = jnp.float32

    e = e_ref[...]
    r = r_ref[...]
    e2 = jnp.sum(e * e, axis=1, keepdims=True)
    r2 = jnp.sum(r * r, axis=1, keepdims=True)
    e2row = e2.reshape(1, _BT)

    m2 = lax.dot_general(r * f32(-2.0), e, (((1,), (1,)), ((), ())),
                         preferred_element_type=f32)
    d2 = (r2 + e2row) + m2

    rv = d2[0:128, :]
    cs = jnp.zeros((128, _BT), f32)
    for s in range(1, _N_CLUSTER // 128):
        ch = d2[s * 128:(s + 1) * 128, :]
        cs = jnp.where(ch < rv, f32(s), cs)
        rv = jnp.minimum(rv, ch)
    rmin = jnp.min(rv, axis=0, keepdims=True)
    jfull = cs * f32(128.0) + lax.broadcasted_iota(
        jnp.int32, (128, _BT), 0).astype(f32)
    ridx = jnp.min(jnp.where(rv == rmin, jfull, _BIG), axis=0)
    idxe_ref[0, 0, :] = ridx.astype(jnp.int32)

    @pl.when(i == 0)
    def _():
        cv_ref[...] = jnp.full((_N_CLUSTER, 128), _BIG, f32)
        ci_ref[...] = jnp.zeros((_N_CLUSTER, 128), f32)

    cv = cv_ref[...]
    ci = ci_ref[...]
    for g in range(_BT // 128):
        blk = d2[:, g * 128:(g + 1) * 128]
        ci = jnp.where(blk < cv, f32(i * (_BT // 128) + g), ci)
        cv = jnp.minimum(cv, blk)
    cv_ref[...] = cv
    ci_ref[...] = ci

    @pl.when(i == _NB - 1)
    def _():
        cmin = jnp.min(cv, axis=1, keepdims=True)
        tfull = ci * f32(128.0) + lax.broadcasted_iota(
            jnp.int32, (_N_CLUSTER, 128), 1).astype(f32)
        cidx = jnp.min(jnp.where(cv == cmin, tfull, _BIG), axis=1)
        idxr_ref[...] = cidx.astype(jnp.int32).reshape(1, _N_CLUSTER)


def _nearest_indices(emb_tensor, repr_tensor):
    idxe, idxr = pl.pallas_call(
        _dist_argmin_kernel,
        grid=(_NB,),
        in_specs=[
            pl.BlockSpec((_BT, _EMB), lambda i: (i, 0)),
            pl.BlockSpec((_N_CLUSTER, _EMB), lambda i: (0, 0)),
        ],
        out_specs=[
            pl.BlockSpec((1, 1, _BT), lambda i: (i, 0, 0)),
            pl.BlockSpec((1, _N_CLUSTER), lambda i: (0, 0)),
        ],
        out_shape=[
            jax.ShapeDtypeStruct((_NB, 1, _BT), jnp.int32),
            jax.ShapeDtypeStruct((1, _N_CLUSTER), jnp.int32),
        ],
        scratch_shapes=[pltpu.VMEM((_N_CLUSTER, 128), jnp.float32),
                        pltpu.VMEM((_N_CLUSTER, 128), jnp.float32)],
    )(emb_tensor, repr_tensor)
    return idxe.reshape(_N_TOKENS), idxr[0]


def _sc_gather_body(r_hbm, e_hbm, idxe_hbm, idxr_hbm, xrec_hbm, reprx_hbm,
                    r_sh, e_sh, idx1_v, rows1_v, idx2_v, rows2_v,
                    semi, sem, semo):
    sid = lax.axis_index("s")
    wid = sid * _NC + lax.axis_index("c")
    base1 = wid * _B1
    base2 = wid * _B2

    ld1 = pltpu.async_copy(idxe_hbm.at[wid], idx1_v, semi)
    ld2 = pltpu.async_copy(idxr_hbm.at[wid], idx2_v, semi)

    er = _N_TOKENS // _NS
    rr = _N_CLUSTER // _NS
    pltpu.sync_copy(e_hbm.at[pl.ds(sid * er, er)], e_sh.at[pl.ds(sid * er, er)])
    pltpu.sync_copy(r_hbm.at[pl.ds(sid * rr, rr)], r_sh.at[pl.ds(sid * rr, rr)])
    plsc.subcore_barrier()

    ld1.wait()
    ld2.wait()

    copies = []
    for c in range(_B1 // _CHUNK):
        copies.append(pltpu.async_copy(
            r_sh.at[idx1_v.at[c]], rows1_v.at[pl.ds(c * _CHUNK, _CHUNK)], sem))
    copies.append(pltpu.async_copy(e_sh.at[idx2_v], rows2_v, sem))
    for cp in copies:
        cp.wait()

    st1 = pltpu.async_copy(rows1_v, xrec_hbm.at[pl.ds(base1, _B1)], semo)
    st2 = pltpu.async_copy(rows2_v, reprx_hbm.at[pl.ds(base2, _B2)], semo)
    st1.wait()
    st2.wait()


@functools.cache
def _make_sc_gather():
    return pl.kernel(
        _sc_gather_body,
        out_type=[
            jax.ShapeDtypeStruct((_N_TOKENS, _EMB), jnp.float32),
            jax.ShapeDtypeStruct((_N_CLUSTER, _EMB), jnp.float32),
        ],
        mesh=plsc.VectorSubcoreMesh(core_axis_name="c", subcore_axis_name="s",
                                    num_cores=_NC, num_subcores=_NS),
        scratch_types=[
            pltpu.VMEM_SHARED((_N_CLUSTER, _EMB), jnp.float32),
            pltpu.VMEM_SHARED((_N_TOKENS, _EMB), jnp.float32),
            pltpu.VMEM((_B1 // _CHUNK, _CHUNK), jnp.int32),
            pltpu.VMEM((_B1, _EMB), jnp.float32),
            pltpu.VMEM((_B2,), jnp.int32),
            pltpu.VMEM((_B2, _EMB), jnp.float32),
            pltpu.SemaphoreType.DMA,
            pltpu.SemaphoreType.DMA,
            pltpu.SemaphoreType.DMA,
        ],
        compiler_params=pltpu.CompilerParams(use_tc_tiling_on_sc=False),
    )


def kernel(emb_tensor, repr_tensor):
    idx_e, idx_r = _nearest_indices(emb_tensor, repr_tensor)
    idxe_3d = idx_e.reshape(_NW, _B1 // _CHUNK, _CHUNK)
    idxr_2d = idx_r.reshape(_NW, _B2)
    x_recon, repr_x = _make_sc_gather()(repr_tensor, emb_tensor, idxe_3d, idxr_2d)
    return (x_recon, emb_tensor, repr_tensor, repr_x)


# speedup vs baseline: 1.0305x; 1.0305x over previous
import functools

import jax
import jax.numpy as jnp
from jax import lax
from jax.experimental import pallas as pl
from jax.experimental.pallas import tpu as pltpu
from jax.experimental.pallas import tpu_sc as plsc

_N_TOKENS = 8192
_N_CLUSTER = 1024
_EMB = 64

_BT = 2048
_NB = _N_TOKENS // _BT
_NSTRIP = _N_CLUSTER // 128
_NCH = 8
_C = _BT // _NCH

_NC = 2
_NS = 16
_NW = _NC * _NS
_B1 = _N_TOKENS // _NW
_B2 = _N_CLUSTER // _NW
_CHUNK = 32

def _dist_argmin_kernel(e_ref, r_ref, idxe_ref, idxr_ref, cv_ref, ci_ref):
    i = pl.program_id(0)
    _BIG = jnp.float32(1e9)
    f32 = jnp.float32

    e = e_ref[...]
    r = r_ref[...]
    e2 = jnp.sum(e * e, axis=1, keepdims=True)
    r2 = jnp.sum(r * r, axis=1, keepdims=True)
    e2row = e2.reshape(1, _BT)

    @pl.when(i == 0)
    def _():
        cv_ref[...] = jnp.full((_N_CLUSTER, 128), _BIG, f32)
        ci_ref[...] = jnp.zeros((_N_CLUSTER, 128), f32)

    rm2 = r * f32(-2.0)
    row_iota = lax.broadcasted_iota(jnp.int32, (128, 128), 0).astype(f32)
    cv = cv_ref[...]
    ci = ci_ref[...]

    for g in range(_BT // 128):
        eg = e[g * 128:(g + 1) * 128, :]
        m2g = lax.dot_general(rm2, eg, (((1,), (1,)), ((), ())),
                              preferred_element_type=f32)
        d2g = (r2 + e2row[:, g * 128:(g + 1) * 128]) + m2g

        ci = jnp.where(d2g < cv, f32(i * (_BT // 128) + g), ci)
        cv = jnp.minimum(cv, d2g)

        rvg = d2g[0:128, :]
        csg = jnp.zeros((128, 128), f32)
        for s in range(1, _N_CLUSTER // 128):
            chg = d2g[s * 128:(s + 1) * 128, :]
            csg = jnp.where(chg < rvg, f32(s), csg)
            rvg = jnp.minimum(rvg, chg)
        rming = jnp.min(rvg, axis=0, keepdims=True)
        jfullg = csg * f32(128.0) + row_iota
        ridxg = jnp.min(jnp.where(rvg == rming, jfullg, _BIG), axis=0)
        idxe_ref[0, 0, g * 128:(g + 1) * 128] = ridxg.astype(jnp.int32)

    cv_ref[...] = cv
    ci_ref[...] = ci

    @pl.when(i == _NB - 1)
    def _():
        cmin = jnp.min(cv, axis=1, keepdims=True)
        tfull = ci * f32(128.0) + lax.broadcasted_iota(
            jnp.int32, (_N_CLUSTER, 128), 1).astype(f32)
        cidx = jnp.min(jnp.where(cv == cmin, tfull, _BIG), axis=1)
        idxr_ref[...] = cidx.astype(jnp.int32).reshape(1, _N_CLUSTER)


def _nearest_indices(emb_tensor, repr_tensor):
    idxe, idxr = pl.pallas_call(
        _dist_argmin_kernel,
        grid=(_NB,),
        in_specs=[
            pl.BlockSpec((_BT, _EMB), lambda i: (i, 0)),
            pl.BlockSpec((_N_CLUSTER, _EMB), lambda i: (0, 0)),
        ],
        out_specs=[
            pl.BlockSpec((1, 1, _BT), lambda i: (i, 0, 0)),
            pl.BlockSpec((1, _N_CLUSTER), lambda i: (0, 0)),
        ],
        out_shape=[
            jax.ShapeDtypeStruct((_NB, 1, _BT), jnp.int32),
            jax.ShapeDtypeStruct((1, _N_CLUSTER), jnp.int32),
        ],
        scratch_shapes=[pltpu.VMEM((_N_CLUSTER, 128), jnp.float32),
                        pltpu.VMEM((_N_CLUSTER, 128), jnp.float32)],
    )(emb_tensor, repr_tensor)
    return idxe.reshape(_N_TOKENS), idxr[0]


def _sc_gather_body(r_hbm, e_hbm, idxe_hbm, idxr_hbm, xrec_hbm, reprx_hbm,
                    r_sh, e_sh, idx1_v, rows1_v, idx2_v, rows2_v,
                    semi, sem, semo):
    sid = lax.axis_index("s")
    wid = sid * _NC + lax.axis_index("c")
    base1 = wid * _B1
    base2 = wid * _B2

    ld1 = pltpu.async_copy(idxe_hbm.at[wid], idx1_v, semi)
    ld2 = pltpu.async_copy(idxr_hbm.at[wid], idx2_v, semi)

    er = _N_TOKENS // _NS
    rr = _N_CLUSTER // _NS
    pltpu.sync_copy(e_hbm.at[pl.ds(sid * er, er)], e_sh.at[pl.ds(sid * er, er)])
    pltpu.sync_copy(r_hbm.at[pl.ds(sid * rr, rr)], r_sh.at[pl.ds(sid * rr, rr)])
    plsc.subcore_barrier()

    ld1.wait()
    ld2.wait()

    copies = []
    for c in range(_B1 // _CHUNK):
        copies.append(pltpu.async_copy(
            r_sh.at[idx1_v.at[c]], rows1_v.at[pl.ds(c * _CHUNK, _CHUNK)], sem))
    copies.append(pltpu.async_copy(e_sh.at[idx2_v], rows2_v, sem))
    for cp in copies:
        cp.wait()

    st1 = pltpu.async_copy(rows1_v, xrec_hbm.at[pl.ds(base1, _B1)], semo)
    st2 = pltpu.async_copy(rows2_v, reprx_hbm.at[pl.ds(base2, _B2)], semo)
    st1.wait()
    st2.wait()


@functools.cache
def _make_sc_gather():
    return pl.kernel(
        _sc_gather_body,
        out_type=[
            jax.ShapeDtypeStruct((_N_TOKENS, _EMB), jnp.float32),
            jax.ShapeDtypeStruct((_N_CLUSTER, _EMB), jnp.float32),
        ],
        mesh=plsc.VectorSubcoreMesh(core_axis_name="c", subcore_axis_name="s",
                                    num_cores=_NC, num_subcores=_NS),
        scratch_types=[
            pltpu.VMEM_SHARED((_N_CLUSTER, _EMB), jnp.float32),
            pltpu.VMEM_SHARED((_N_TOKENS, _EMB), jnp.float32),
            pltpu.VMEM((_B1 // _CHUNK, _CHUNK), jnp.int32),
            pltpu.VMEM((_B1, _EMB), jnp.float32),
            pltpu.VMEM((_B2,), jnp.int32),
            pltpu.VMEM((_B2, _EMB), jnp.float32),
            pltpu.SemaphoreType.DMA,
            pltpu.SemaphoreType.DMA,
            pltpu.SemaphoreType.DMA,
        ],
        compiler_params=pltpu.CompilerParams(use_tc_tiling_on_sc=False),
    )


def kernel(emb_tensor, repr_tensor):
    idx_e, idx_r = _nearest_indices(emb_tensor, repr_tensor)
    idxe_3d = idx_e.reshape(_NW, _B1 // _CHUNK, _CHUNK)
    idxr_2d = idx_r.reshape(_NW, _B2)
    x_recon, repr_x = _make_sc_gather()(repr_tensor, emb_tensor, idxe_3d, idxr_2d)
    return (x_recon, emb_tensor, repr_tensor, repr_x)


# speedup vs baseline: 1.0861x; 1.0539x over previous
import functools

import jax
import jax.numpy as jnp
from jax import lax
from jax.experimental import pallas as pl
from jax.experimental.pallas import tpu as pltpu
from jax.experimental.pallas import tpu_sc as plsc

_N_TOKENS = 8192
_N_CLUSTER = 1024
_EMB = 64

_BT = 8192
_NB = _N_TOKENS // _BT
_NSTRIP = _N_CLUSTER // 128
_NCH = 8
_C = _BT // _NCH

_NC = 2
_NS = 16
_NW = _NC * _NS
_B1 = _N_TOKENS // _NW
_B2 = _N_CLUSTER // _NW
_CHUNK = 32

def _dist_argmin_kernel(e_ref, r_ref, idxe_ref, idxr_ref, cv_ref, ci_ref):
    i = pl.program_id(0)
    _BIG = jnp.float32(1e9)
    f32 = jnp.float32

    e = e_ref[...]
    r = r_ref[...]
    e2 = jnp.sum(e * e, axis=1, keepdims=True)
    r2 = jnp.sum(r * r, axis=1, keepdims=True)
    e2row = e2.reshape(1, _BT)

    @pl.when(i == 0)
    def _():
        cv_ref[...] = jnp.full((_N_CLUSTER, 128), _BIG, f32)
        ci_ref[...] = jnp.zeros((_N_CLUSTER, 128), f32)

    rm2 = r * f32(-2.0)
    row_iota = lax.broadcasted_iota(jnp.int32, (128, 128), 0).astype(f32)
    cv = cv_ref[...]
    ci = ci_ref[...]

    for g in range(_BT // 128):
        eg = e[g * 128:(g + 1) * 128, :]
        m2g = lax.dot_general(rm2, eg, (((1,), (1,)), ((), ())),
                              preferred_element_type=f32)
        d2g = (r2 + e2row[:, g * 128:(g + 1) * 128]) + m2g

        ci = jnp.where(d2g < cv, f32(i * (_BT // 128) + g), ci)
        cv = jnp.minimum(cv, d2g)

        rvg = d2g[0:128, :]
        csg = jnp.zeros((128, 128), f32)
        for s in range(1, _N_CLUSTER // 128):
            chg = d2g[s * 128:(s + 1) * 128, :]
            csg = jnp.where(chg < rvg, f32(s), csg)
            rvg = jnp.minimum(rvg, chg)
        rming = jnp.min(rvg, axis=0, keepdims=True)
        jfullg = csg * f32(128.0) + row_iota
        ridxg = jnp.min(jnp.where(rvg == rming, jfullg, _BIG), axis=0)
        idxe_ref[0, 0, g * 128:(g + 1) * 128] = ridxg.astype(jnp.int32)

    cv_ref[...] = cv
    ci_ref[...] = ci

    @pl.when(i == _NB - 1)
    def _():
        cmin = jnp.min(cv, axis=1, keepdims=True)
        tfull = ci * f32(128.0) + lax.broadcasted_iota(
            jnp.int32, (_N_CLUSTER, 128), 1).astype(f32)
        cidx = jnp.min(jnp.where(cv == cmin, tfull, _BIG), axis=1)
        idxr_ref[...] = cidx.astype(jnp.int32).reshape(1, _N_CLUSTER)


def _nearest_indices(emb_tensor, repr_tensor):
    idxe, idxr = pl.pallas_call(
        _dist_argmin_kernel,
        grid=(_NB,),
        in_specs=[
            pl.BlockSpec((_BT, _EMB), lambda i: (i, 0)),
            pl.BlockSpec((_N_CLUSTER, _EMB), lambda i: (0, 0)),
        ],
        out_specs=[
            pl.BlockSpec((1, 1, _BT), lambda i: (i, 0, 0)),
            pl.BlockSpec((1, _N_CLUSTER), lambda i: (0, 0)),
        ],
        out_shape=[
            jax.ShapeDtypeStruct((_NB, 1, _BT), jnp.int32),
            jax.ShapeDtypeStruct((1, _N_CLUSTER), jnp.int32),
        ],
        scratch_shapes=[pltpu.VMEM((_N_CLUSTER, 128), jnp.float32),
                        pltpu.VMEM((_N_CLUSTER, 128), jnp.float32)],
    )(emb_tensor, repr_tensor)
    return idxe.reshape(_N_TOKENS), idxr[0]


def _sc_gather_body(r_hbm, e_hbm, idxe_hbm, idxr_hbm, xrec_hbm, reprx_hbm,
                    r_sh, e_sh, idx1_v, rows1_v, idx2_v, rows2_v,
                    semi, sem, semo):
    sid = lax.axis_index("s")
    wid = sid * _NC + lax.axis_index("c")
    base1 = wid * _B1
    base2 = wid * _B2

    ld1 = pltpu.async_copy(idxe_hbm.at[wid], idx1_v, semi)
    ld2 = pltpu.async_copy(idxr_hbm.at[wid], idx2_v, semi)

    er = _N_TOKENS // _NS
    rr = _N_CLUSTER // _NS
    pltpu.sync_copy(e_hbm.at[pl.ds(sid * er, er)], e_sh.at[pl.ds(sid * er, er)])
    pltpu.sync_copy(r_hbm.at[pl.ds(sid * rr, rr)], r_sh.at[pl.ds(sid * rr, rr)])
    plsc.subcore_barrier()

    ld1.wait()
    ld2.wait()

    copies = []
    for c in range(_B1 // _CHUNK):
        copies.append(pltpu.async_copy(
            r_sh.at[idx1_v.at[c]], rows1_v.at[pl.ds(c * _CHUNK, _CHUNK)], sem))
    copies.append(pltpu.async_copy(e_sh.at[idx2_v], rows2_v, sem))
    for cp in copies:
        cp.wait()

    st1 = pltpu.async_copy(rows1_v, xrec_hbm.at[pl.ds(base1, _B1)], semo)
    st2 = pltpu.async_copy(rows2_v, reprx_hbm.at[pl.ds(base2, _B2)], semo)
    st1.wait()
    st2.wait()


@functools.cache
def _make_sc_gather():
    return pl.kernel(
        _sc_gather_body,
        out_type=[
            jax.ShapeDtypeStruct((_N_TOKENS, _EMB), jnp.float32),
            jax.ShapeDtypeStruct((_N_CLUSTER, _EMB), jnp.float32),
        ],
        mesh=plsc.VectorSubcoreMesh(core_axis_name="c", subcore_axis_name="s",
                                    num_cores=_NC, num_subcores=_NS),
        scratch_types=[
            pltpu.VMEM_SHARED((_N_CLUSTER, _EMB), jnp.float32),
            pltpu.VMEM_SHARED((_N_TOKENS, _EMB), jnp.float32),
            pltpu.VMEM((_B1 // _CHUNK, _CHUNK), jnp.int32),
            pltpu.VMEM((_B1, _EMB), jnp.float32),
            pltpu.VMEM((_B2,), jnp.int32),
            pltpu.VMEM((_B2, _EMB), jnp.float32),
            pltpu.SemaphoreType.DMA,
            pltpu.SemaphoreType.DMA,
            pltpu.SemaphoreType.DMA,
        ],
        compiler_params=pltpu.CompilerParams(use_tc_tiling_on_sc=False),
    )


def kernel(emb_tensor, repr_tensor):
    idx_e, idx_r = _nearest_indices(emb_tensor, repr_tensor)
    idxe_3d = idx_e.reshape(_NW, _B1 // _CHUNK, _CHUNK)
    idxr_2d = idx_r.reshape(_NW, _B2)
    x_recon, repr_x = _make_sc_gather()(repr_tensor, emb_tensor, idxe_3d, idxr_2d)
    return (x_recon, emb_tensor, repr_tensor, repr_x)
